# baseline (speedup 1.0000x reference)
import jax
import jax.numpy as jnp
from jax import lax
from jax.experimental import pallas as pl
from jax.experimental.pallas import tpu as pltpu
from jax.experimental.pallas import tpu_sc as plsc

B = 16384
D = 1024
LANES = 16
NC = 2
NS = 16
NW = NC * NS
B_PER_W = B // NW
N_GRP = B_PER_W // LANES


def _body(idx_hbm, table_hbm, out_hbm, idx_v, table_v, sem):
    sid = lax.axis_index("s")
    wid = sid * NC + lax.axis_index("c")
    base = wid * B_PER_W

    pltpu.sync_copy(table_hbm, table_v)
    pltpu.sync_copy(idx_hbm.at[pl.ds(base, B_PER_W)], idx_v)

    def grp_body(g, carry):
        tvec = idx_v[pl.ds(g * LANES, LANES)]
        for j in range(LANES):
            t = tvec[j]
            pltpu.async_copy(table_v.at[t], out_hbm.at[base + g * LANES + j], sem)
        return carry

    lax.fori_loop(0, N_GRP, grp_body, 0)

    def drain_body(i, carry):
        pltpu.make_async_copy(table_hbm.at[0], table_v.at[0], sem).wait()
        return carry

    lax.fori_loop(0, B_PER_W, drain_body, 0)


_sc_lookup = pl.kernel(
    _body,
    out_type=jax.ShapeDtypeStruct((B, D), jnp.float32),
    mesh=plsc.VectorSubcoreMesh(core_axis_name="c", subcore_axis_name="s"),
    scratch_types=[
        pltpu.VMEM((B_PER_W,), jnp.int32),
        pltpu.VMEM((2, D), jnp.float32),
        pltpu.SemaphoreType.DMA,
    ],
)


def kernel(domain_idx, embed_weight):
    return _sc_lookup(domain_idx.astype(jnp.int32), embed_weight)


# speedup vs baseline: 1.0458x; 1.0458x over previous
import jax
import jax.numpy as jnp
from jax import lax
from jax.experimental import pallas as pl
from jax.experimental.pallas import tpu as pltpu
from jax.experimental.pallas import tpu_sc as plsc

B = 16384
D = 1024
LANES = 16
NC = 2
NS = 16
NW = NC * NS
B_PER_W = B // NW
N_GRP = B_PER_W // LANES
DRAIN_ROWS = 16


def _body(idx_hbm, table_hbm, out_hbm, idx_v, table_v, drain_v, sem):
    sid = lax.axis_index("s")
    wid = sid * NC + lax.axis_index("c")
    base = wid * B_PER_W

    pltpu.sync_copy(table_hbm, table_v)
    pltpu.sync_copy(idx_hbm.at[pl.ds(base, B_PER_W)], idx_v)

    def grp_body(g, carry):
        row = base + g * LANES
        tvec = idx_v[pl.ds(g * LANES, LANES)]
        for j in range(LANES):
            pltpu.async_copy(table_v.at[tvec[j]], out_hbm.at[row + j], sem)
        return carry

    lax.fori_loop(0, N_GRP, grp_body, 0)

    def drain_body(i, carry):
        pltpu.make_async_copy(out_hbm.at[pl.ds(base, DRAIN_ROWS)], drain_v, sem).wait()
        return carry

    lax.fori_loop(0, B_PER_W // DRAIN_ROWS, drain_body, 0)


_sc_lookup = pl.kernel(
    _body,
    out_type=jax.ShapeDtypeStruct((B, D), jnp.float32),
    mesh=plsc.VectorSubcoreMesh(core_axis_name="c", subcore_axis_name="s"),
    scratch_types=[
        pltpu.VMEM((B_PER_W,), jnp.int32),
        pltpu.VMEM((2, D), jnp.float32),
        pltpu.VMEM((DRAIN_ROWS, D), jnp.float32),
        pltpu.SemaphoreType.DMA,
    ],
)


def kernel(domain_idx, embed_weight):
    return _sc_lookup(domain_idx.astype(jnp.int32), embed_weight)
